# baseline (speedup 1.0000x reference)
import jax
import jax.numpy as jnp
from jax.experimental import pallas as pl
from jax.experimental.pallas import tpu as pltpu


def _conv3x3s2_kernel(x_ref, wa_ref, wb_ref, b_ref, o_ref):
    _, Hh, _, Wh, C2 = x_ref.shape
    C = C2 // 2
    Ho, Wo = Hh - 1, Wh - 1
    acc = jnp.zeros((Ho * Wo, o_ref.shape[-1]), jnp.float32)
    for ky in range(3):
        h0, r = ky // 2, ky % 2
        lhs_a = x_ref[0, h0:h0 + Ho, r, 0:Wo, :].reshape(Ho * Wo, C2)
        acc = acc + jnp.dot(lhs_a, wa_ref[ky],
                            preferred_element_type=jnp.float32)
        lhs_b = x_ref[0, h0:h0 + Ho, r, 1:Wo + 1, 0:C].reshape(Ho * Wo, C)
        acc = acc + jnp.dot(lhs_b, wb_ref[ky],
                            preferred_element_type=jnp.float32)
    o_ref[0] = (acc + b_ref[...]).astype(o_ref.dtype)


def kernel(x_nchw, w_oihw, bias):
    N, C, H, W = x_nchw.shape
    Cout = w_oihw.shape[0]
    Ho = (H - 2) // 2 + 1
    Wo = (W - 2) // 2 + 1
    Hp, Wp = 2 * Ho + 2, 2 * Wo + 2
    Hh, Wh = Hp // 2, Wp // 2

    x = jnp.transpose(x_nchw, (0, 2, 3, 1)).astype(jnp.bfloat16)
    x = jnp.pad(x, ((0, 0), (0, Hp - H), (0, Wp - W), (0, 0)))
    x = x.reshape(N, Hh, 2, Wh, 2 * C)

    wt = jnp.transpose(w_oihw, (2, 3, 1, 0)).astype(jnp.bfloat16)
    wa = jnp.concatenate([wt[:, 0], wt[:, 1]], axis=1)
    wb = wt[:, 2]
    b2 = bias.reshape(1, Cout).astype(jnp.float32)

    out = pl.pallas_call(
        _conv3x3s2_kernel,
        out_shape=jax.ShapeDtypeStruct((N, Ho * Wo, Cout), jnp.bfloat16),
        grid=(N,),
        in_specs=[
            pl.BlockSpec((1, Hh, 2, Wh, 2 * C), lambda n: (n, 0, 0, 0, 0)),
            pl.BlockSpec((3, 2 * C, Cout), lambda n: (0, 0, 0)),
            pl.BlockSpec((3, C, Cout), lambda n: (0, 0, 0)),
            pl.BlockSpec((1, Cout), lambda n: (0, 0)),
        ],
        out_specs=pl.BlockSpec((1, Ho * Wo, Cout), lambda n: (n, 0, 0)),
        compiler_params=pltpu.CompilerParams(
            dimension_semantics=("parallel",),
            vmem_limit_bytes=64 * 1024 * 1024),
    )(x, wa, wb, b2)

    out = out.reshape(N, Ho, Wo, Cout)
    return jnp.transpose(out, (0, 3, 1, 2)).astype(jnp.float32)


# speedup vs baseline: 1.3030x; 1.3030x over previous
import jax
import jax.numpy as jnp
from jax.experimental import pallas as pl
from jax.experimental.pallas import tpu as pltpu


def _conv3x3s2_kernel(x_ref, wa_ref, wb_ref, b_ref, o_ref):
    _, Hh, _, Wh, C2 = x_ref.shape
    C = C2 // 2
    Ho, Wo = Hh, Wh
    dt = x_ref.dtype
    acc = jnp.zeros((Ho * Wo, o_ref.shape[-1]), jnp.float32)
    zrow_a = jnp.zeros((1, Wo, C2), dt)
    zcol = jnp.zeros((Ho, 1, C), dt)
    for ky in range(3):
        h0, r = ky // 2, ky % 2
        if h0 == 0:
            lhs_a = x_ref[0, :, r, :, :]
        else:
            lhs_a = jnp.concatenate([x_ref[0, 1:Hh, r, :, :], zrow_a], axis=0)
        acc = acc + jnp.dot(lhs_a.reshape(Ho * Wo, C2), wa_ref[ky],
                            preferred_element_type=jnp.float32)
        if h0 == 0:
            core = x_ref[0, :, r, 1:Wh, 0:C]
        else:
            core = jnp.concatenate(
                [x_ref[0, 1:Hh, r, 1:Wh, 0:C],
                 jnp.zeros((1, Wh - 1, C), dt)], axis=0)
        lhs_b = jnp.concatenate([core, zcol], axis=1)
        acc = acc + jnp.dot(lhs_b.reshape(Ho * Wo, C), wb_ref[ky],
                            preferred_element_type=jnp.float32)
    o_ref[0] = acc + b_ref[...]


def kernel(x_nchw, w_oihw, bias):
    N, C, H, W = x_nchw.shape
    Cout = w_oihw.shape[0]
    Ho = (H - 2) // 2 + 1
    Wo = (W - 2) // 2 + 1
    Hh, Wh = Ho, Wo

    x = jnp.transpose(x_nchw, (0, 2, 3, 1)).astype(jnp.bfloat16)
    x = x.reshape(N, Hh, 2, Wh, 2 * C)

    wt = jnp.transpose(w_oihw, (2, 3, 1, 0)).astype(jnp.bfloat16)
    wa = jnp.concatenate([wt[:, 0], wt[:, 1]], axis=1)
    wb = wt[:, 2]
    b2 = bias.reshape(1, Cout).astype(jnp.float32)

    out = pl.pallas_call(
        _conv3x3s2_kernel,
        out_shape=jax.ShapeDtypeStruct((N, Ho * Wo, Cout), jnp.float32),
        grid=(N,),
        in_specs=[
            pl.BlockSpec((1, Hh, 2, Wh, 2 * C), lambda n: (n, 0, 0, 0, 0)),
            pl.BlockSpec((3, 2 * C, Cout), lambda n: (0, 0, 0)),
            pl.BlockSpec((3, C, Cout), lambda n: (0, 0, 0)),
            pl.BlockSpec((1, Cout), lambda n: (0, 0)),
        ],
        out_specs=pl.BlockSpec((1, Ho * Wo, Cout), lambda n: (n, 0, 0)),
        compiler_params=pltpu.CompilerParams(
            dimension_semantics=("parallel",),
            vmem_limit_bytes=64 * 1024 * 1024),
    )(x, wa, wb, b2)

    out = out.reshape(N, Ho, Wo, Cout)
    return jnp.transpose(out, (0, 3, 1, 2))
